# baseline (speedup 1.0000x reference)
import functools
import math

import jax
import jax.numpy as jnp
from jax import lax
from jax.experimental import pallas as pl
from jax.experimental.pallas import tpu as pltpu
from jax.experimental.pallas import tpu_sc as plsc

_B, _L, _V, _D = 1024, 512, 256, 128

_NC, _NS, _LANES = 2, 16, 16
_NW = _NC * _NS
_RPW = _B // _NW
_GROUPS = _RPW // _LANES
_VCHUNKS = _V // _LANES


def _sc_hist_body(ids_hbm, ht_hbm, ids_v, hist_v):
    wid = lax.axis_index("s") * _NC + lax.axis_index("c")
    base = wid * _RPW
    pltpu.sync_copy(ids_hbm.at[pl.ds(base, _RPW)], ids_v)

    lane = lax.iota(jnp.int32, 16)
    zeros = jnp.zeros((_LANES,), jnp.float32)
    ones = jnp.ones((_LANES,), jnp.float32)

    @plsc.parallel_loop(0, _V * _GROUPS, unroll=8)
    def _zero(i):
        hist_v[i // _GROUPS, pl.ds((i % _GROUPS) * _LANES, _LANES)] = zeros

    @plsc.parallel_loop(0, _L, unroll=8)
    def _pos(p):
        for g in range(_GROUPS):
            row = g * _LANES + lane
            q = (p + lane) & (_L - 1)
            ids16 = plsc.load_gather(ids_v, [row, q])
            plsc.addupdate_scatter(hist_v, [ids16, row], ones)

    pltpu.sync_copy(hist_v, ht_hbm.at[wid])


_sc_hist = functools.partial(
    pl.kernel,
    mesh=plsc.VectorSubcoreMesh(core_axis_name="c", subcore_axis_name="s",
                                num_cores=_NC),
    compiler_params=pltpu.CompilerParams(needs_layout_passes=False),
    out_type=jax.ShapeDtypeStruct((_NW, _V, _RPW), jnp.float32),
    scratch_types=[
        pltpu.VMEM((_RPW, _L), jnp.int32),
        pltpu.VMEM((_V, _RPW), jnp.float32),
    ],
)(_sc_hist_body)


_TCG = 4
_WPB = _NW // _TCG


def _tc_mlp_body(ht_ref, tab_ref, w1_ref, b1_ref, w2_ref, b2_ref, o_ref,
                 pooled_ref):
    tab = tab_ref[...]
    for w in range(_WPB):
        pooled_ref[pl.ds(w * _RPW, _RPW), :] = lax.dot_general(
            ht_ref[w], tab,
            dimension_numbers=(((0,), (0,)), ((), ())),
            preferred_element_type=jnp.float32)
    pooled = pooled_ref[...] * (1.0 / _L)
    x = jnp.dot(pooled, w1_ref[...],
                preferred_element_type=jnp.float32) + b1_ref[...]
    x = 0.5 * x * (1.0 + lax.erf(x * (1.0 / math.sqrt(2.0))))
    o_ref[...] = jnp.dot(x, w2_ref[...],
                         preferred_element_type=jnp.float32) + b2_ref[...]


def kernel(prompt_ids, table, W1, b1, W2, b2):
    ht = _sc_hist(prompt_ids.astype(jnp.int32))
    out = pl.pallas_call(
        _tc_mlp_body,
        grid=(_TCG,),
        in_specs=[
            pl.BlockSpec((_WPB, _V, _RPW), lambda i: (i, 0, 0)),
            pl.BlockSpec((_V, _D), lambda i: (0, 0)),
            pl.BlockSpec((_D, _D), lambda i: (0, 0)),
            pl.BlockSpec((1, _D), lambda i: (0, 0)),
            pl.BlockSpec((_D, _D), lambda i: (0, 0)),
            pl.BlockSpec((1, _D), lambda i: (0, 0)),
        ],
        out_specs=pl.BlockSpec((_B // _TCG, _D), lambda i: (i, 0)),
        out_shape=jax.ShapeDtypeStruct((_B, _D), jnp.float32),
        scratch_shapes=[pltpu.VMEM((_B // _TCG, _D), jnp.float32)],
    )(ht, table, W1, b1.reshape(1, _D), W2, b2.reshape(1, _D))
    return out


# speedup vs baseline: 1.0867x; 1.0867x over previous
import functools
import math

import jax
import jax.numpy as jnp
from jax import lax
from jax.experimental import pallas as pl
from jax.experimental.pallas import tpu as pltpu
from jax.experimental.pallas import tpu_sc as plsc

_B, _L, _V, _D = 1024, 512, 256, 128

_NC, _NS, _LANES = 2, 16, 16
_NW = _NC * _NS
_RPW = _B // _NW
_GROUPS = _RPW // _LANES
_VCHUNKS = _V // _LANES


def _sc_hist_body(ids_hbm, h_hbm, ids_v, hist_v):
    wid = lax.axis_index("s") * _NC + lax.axis_index("c")
    base = wid * _RPW
    pltpu.sync_copy(ids_hbm.at[pl.ds(base, _RPW)], ids_v)

    lane = lax.iota(jnp.int32, 16)
    zeros = jnp.zeros((_LANES,), jnp.float32)
    ones = jnp.ones((_LANES,), jnp.float32)

    @plsc.parallel_loop(0, _RPW * _VCHUNKS, unroll=8)
    def _zero(i):
        hist_v[i // _VCHUNKS, pl.ds((i % _VCHUNKS) * _LANES, _LANES)] = zeros

    @plsc.parallel_loop(0, _L, unroll=8)
    def _pos(p):
        for g in range(_GROUPS):
            row = g * _LANES + lane
            q = (p + lane) & (_L - 1)
            ids16 = plsc.load_gather(ids_v, [row, q])
            plsc.addupdate_scatter(hist_v, [row, ids16], ones)

    pltpu.sync_copy(hist_v, h_hbm.at[pl.ds(base, _RPW)])


_sc_hist = functools.partial(
    pl.kernel,
    mesh=plsc.VectorSubcoreMesh(core_axis_name="c", subcore_axis_name="s",
                                num_cores=_NC),
    compiler_params=pltpu.CompilerParams(needs_layout_passes=False),
    out_type=jax.ShapeDtypeStruct((_B, _V), jnp.float32),
    scratch_types=[
        pltpu.VMEM((_RPW, _L), jnp.int32),
        pltpu.VMEM((_RPW, _V), jnp.float32),
    ],
)(_sc_hist_body)


def _tc_mlp_body(h_ref, tab_ref, w1_ref, b1_ref, w2_ref, b2_ref, o_ref):
    pooled = jnp.dot(h_ref[...], tab_ref[...],
                     preferred_element_type=jnp.float32) * (1.0 / _L)
    x = jnp.dot(pooled, w1_ref[...],
                preferred_element_type=jnp.float32) + b1_ref[...]
    x = 0.5 * x * (1.0 + lax.erf(x * (1.0 / math.sqrt(2.0))))
    o_ref[...] = jnp.dot(x, w2_ref[...],
                         preferred_element_type=jnp.float32) + b2_ref[...]


def kernel(prompt_ids, table, W1, b1, W2, b2):
    h = _sc_hist(prompt_ids.astype(jnp.int32))
    out = pl.pallas_call(
        _tc_mlp_body,
        out_shape=jax.ShapeDtypeStruct((_B, _D), jnp.float32),
    )(h, table, W1, b1.reshape(1, _D), W2, b2.reshape(1, _D))
    return out
